# baseline (speedup 1.0000x reference)
import functools

import jax
import jax.numpy as jnp
from jax import lax
from jax.experimental import pallas as pl
from jax.experimental.pallas import tpu as pltpu
from jax.experimental.pallas import tpu_sc as plsc

B, L, K, D = 8, 1024, 16, 128
R = 128
BH = B

NC, NS = 2, 16
NW = NC * NS
IDX_COLS = 128
TOTAL_H = BH * L * K
IDX_ROWS_H = TOTAL_H // IDX_COLS
ROWS_PER_W = IDX_ROWS_H // NW


def _make_topk_body(b0):
    def _topk_body(ci_ref, ctT_ref, ctr8_ref, ax_ref, coords_ref, nbr_ref,
                   d_ref, nc_ref):
        b = pl.program_id(0)
        cxj = ctT_ref[0:1, :]
        cyj = ctT_ref[1:2, :]
        czj = ctT_ref[2:3, :]
        cxi = ci_ref[:, 0:1]
        cyi = ci_ref[:, 1:2]
        czi = ci_ref[:, 2:3]
        d0 = (cxi - cxj) ** 2
        d1 = (cyi - cyj) ** 2
        d2 = (czi - czj) ** 2
        d_ref[...] = d0 + d1 + d2
        iota_f = lax.broadcasted_iota(jnp.int32, (R, L), 1).astype(jnp.float32)
        ctr8 = ctr8_ref[...]
        ctr8_hi = ctr8.astype(jnp.bfloat16).astype(jnp.float32)
        ctr16 = jnp.concatenate([ctr8_hi, ctr8 - ctr8_hi], axis=1)
        dn = (((1,), (0,)), ((), ()))
        lf = jnp.float32(L)
        for k in range(K):
            d = d_ref[...]
            m = jnp.min(d, axis=1, keepdims=True)
            cand = jnp.where(d == m, iota_f, lf)
            idx_f = jnp.min(cand, axis=1, keepdims=True)
            sel = cand == idx_f
            if k < K - 1:
                d_ref[...] = jnp.where(sel, jnp.float32(jnp.inf), d)
            selb = jnp.where(sel, jnp.float32(1.0), jnp.float32(0.0))
            nc16 = lax.dot_general(selb, ctr16, dn,
                                   preferred_element_type=jnp.float32)
            nc_ref[:, 3 * k:3 * k + 3] = nc16[:, 0:3] + nc16[:, 8:11]
            nbr_ref[:, k:k + 1] = idx_f.astype(jnp.int32) + (b + b0) * L

        KC = 3 * K
        lane = lax.broadcasted_iota(jnp.int32, (R, KC), 1)
        lmod = lax.rem(lane, 3)
        ci_rep = jnp.where(lmod == 0, cxi, jnp.where(lmod == 1, cyi, czi))
        delta = nc_ref[...] - ci_rep
        rows48 = lax.broadcasted_iota(jnp.int32, (KC, KC), 0)
        cols48 = lax.broadcasted_iota(jnp.int32, (KC, KC), 1)
        same_k = (rows48 // 3) == (cols48 // 3)
        acc = jnp.zeros((R, KC), jnp.float32)
        for a in range(3):
            a0 = ax_ref[:, 3 * a:3 * a + 1]
            a1 = ax_ref[:, 3 * a + 1:3 * a + 2]
            a2 = ax_ref[:, 3 * a + 2:3 * a + 3]
            a_rep = jnp.where(lmod == 0, a0, jnp.where(lmod == 1, a1, a2))
            prod = delta * a_rep
            prod_hi = prod.astype(jnp.bfloat16).astype(jnp.float32)
            prod_lo = prod - prod_hi
            s_a = jnp.where(same_k & (lax.rem(cols48, 3) == a),
                            jnp.float32(1.0), jnp.float32(0.0))
            acc = (acc
                   + lax.dot_general(prod_hi, s_a, dn,
                                     preferred_element_type=jnp.float32)
                   + lax.dot_general(prod_lo, s_a, dn,
                                     preferred_element_type=jnp.float32))
        coords_ref[...] = acc

    return _topk_body


def _topk_call(centers, centers_t, centers8, axes9, b0):
    return pl.pallas_call(
        _make_topk_body(b0),
        grid=(BH, L // R),
        in_specs=[
            pl.BlockSpec((None, R, 3), lambda b, r: (b, r, 0)),
            pl.BlockSpec((None, 3, L), lambda b, r: (b, 0, 0)),
            pl.BlockSpec((None, L, 8), lambda b, r: (b, 0, 0)),
            pl.BlockSpec((None, R, 9), lambda b, r: (b, r, 0)),
        ],
        out_specs=[
            pl.BlockSpec((None, R, 3 * K), lambda b, r: (b, r, 0)),
            pl.BlockSpec((None, R, K), lambda b, r: (b, r, 0)),
        ],
        out_shape=[
            jax.ShapeDtypeStruct((BH, L, 3 * K), jnp.float32),
            jax.ShapeDtypeStruct((BH, L, K), jnp.int32),
        ],
        scratch_shapes=[
            pltpu.VMEM((R, L), jnp.float32),
            pltpu.VMEM((R, 3 * K), jnp.float32),
        ],
    )(centers, centers_t, centers8, axes9)


@functools.cache
def _gather_kernel_build():
    @functools.partial(
        pl.kernel,
        mesh=plsc.VectorSubcoreMesh(core_axis_name="c", subcore_axis_name="s"),
        out_type=jax.ShapeDtypeStruct((TOTAL_H, D), jnp.float32),
        scratch_types=[
            pltpu.VMEM((ROWS_PER_W, IDX_COLS), jnp.int32),
            pltpu.VMEM((IDX_COLS, D), jnp.float32),
            pltpu.SemaphoreType.DMA,
        ],
    )
    def _gather_kernel(table_hbm, idx_hbm, out_hbm, idx_v, rows_v, sem):
        wid = lax.axis_index("s") * NC + lax.axis_index("c")
        base = wid * ROWS_PER_W
        pltpu.sync_copy(idx_hbm.at[pl.ds(base, ROWS_PER_W)], idx_v)

        def body(j, carry):
            pltpu.async_copy(table_hbm.at[idx_v.at[j]], rows_v, sem).wait()
            pltpu.sync_copy(
                rows_v, out_hbm.at[pl.ds((base + j) * IDX_COLS, IDX_COLS)])
            return carry

        lax.fori_loop(0, ROWS_PER_W, body, 0)

    return _gather_kernel


def kernel(frame, attr):
    centers = frame[:, :, 0, :]
    centers_t = jnp.transpose(centers, (0, 2, 1))
    centers8 = jnp.concatenate(
        [centers, jnp.zeros((B, L, 5), jnp.float32)], axis=-1)
    axes9 = frame[:, :, 1:4, :].reshape(B, L, 9)
    table = attr.reshape(B * L, D)
    gather = _gather_kernel_build()

    coords, nbr = _topk_call(centers, centers_t, centers8, axes9, 0)
    gathered = gather(table, nbr.reshape(IDX_ROWS_H, IDX_COLS))

    coords4 = coords.reshape(B, L, K, 3)
    attrs4 = gathered.reshape(B, L, K, D)
    return jnp.concatenate([coords4, attrs4], axis=-1)


# speedup vs baseline: 1.2718x; 1.2718x over previous
import functools

import jax
import jax.numpy as jnp
from jax import lax
from jax.experimental import pallas as pl
from jax.experimental.pallas import tpu as pltpu
from jax.experimental.pallas import tpu_sc as plsc

B, L, K, D = 8, 1024, 16, 128
R = 256
BH = B

NC, NS = 2, 16
NW = NC * NS
IDX_COLS = 128
TOTAL_H = BH * L * K
IDX_ROWS_H = TOTAL_H // IDX_COLS
ROWS_PER_W = IDX_ROWS_H // NW


def _make_topk_body(b0):
    def _topk_body(ci_ref, ctT_ref, ctr8_ref, ax_ref, coords_ref, nbr_ref,
                   d_ref, nc_ref):
        b = pl.program_id(0)
        cxj = ctT_ref[0:1, :]
        cyj = ctT_ref[1:2, :]
        czj = ctT_ref[2:3, :]
        cxi = ci_ref[:, 0:1]
        cyi = ci_ref[:, 1:2]
        czi = ci_ref[:, 2:3]
        d0 = (cxi - cxj) ** 2
        d1 = (cyi - cyj) ** 2
        d2 = (czi - czj) ** 2
        d_ref[...] = d0 + d1 + d2
        iota_f = lax.broadcasted_iota(jnp.int32, (R, L), 1).astype(jnp.float32)
        ctr8 = ctr8_ref[...]
        ctr8_hi = ctr8.astype(jnp.bfloat16).astype(jnp.float32)
        ctr16 = jnp.concatenate([ctr8_hi, ctr8 - ctr8_hi], axis=1)
        dn = (((1,), (0,)), ((), ()))
        lf = jnp.float32(L)
        for k in range(K):
            d = d_ref[...]
            m = jnp.min(d, axis=1, keepdims=True)
            cand = jnp.where(d == m, iota_f, lf)
            idx_f = jnp.min(cand, axis=1, keepdims=True)
            sel = cand == idx_f
            if k < K - 1:
                d_ref[...] = jnp.where(sel, jnp.float32(jnp.inf), d)
            selb = jnp.where(sel, jnp.float32(1.0), jnp.float32(0.0))
            nc16 = lax.dot_general(selb, ctr16, dn,
                                   preferred_element_type=jnp.float32)
            nc_ref[:, 3 * k:3 * k + 3] = nc16[:, 0:3] + nc16[:, 8:11]
            nbr_ref[:, k:k + 1] = idx_f.astype(jnp.int32) + (b + b0) * L

        KC = 3 * K
        lane = lax.broadcasted_iota(jnp.int32, (R, KC), 1)
        lmod = lax.rem(lane, 3)
        ci_rep = jnp.where(lmod == 0, cxi, jnp.where(lmod == 1, cyi, czi))
        delta = nc_ref[...] - ci_rep
        rows48 = lax.broadcasted_iota(jnp.int32, (KC, KC), 0)
        cols48 = lax.broadcasted_iota(jnp.int32, (KC, KC), 1)
        same_k = (rows48 // 3) == (cols48 // 3)
        acc = jnp.zeros((R, KC), jnp.float32)
        for a in range(3):
            a0 = ax_ref[:, 3 * a:3 * a + 1]
            a1 = ax_ref[:, 3 * a + 1:3 * a + 2]
            a2 = ax_ref[:, 3 * a + 2:3 * a + 3]
            a_rep = jnp.where(lmod == 0, a0, jnp.where(lmod == 1, a1, a2))
            prod = delta * a_rep
            prod_hi = prod.astype(jnp.bfloat16).astype(jnp.float32)
            prod_lo = prod - prod_hi
            s_a = jnp.where(same_k & (lax.rem(cols48, 3) == a),
                            jnp.float32(1.0), jnp.float32(0.0))
            acc = (acc
                   + lax.dot_general(prod_hi, s_a, dn,
                                     preferred_element_type=jnp.float32)
                   + lax.dot_general(prod_lo, s_a, dn,
                                     preferred_element_type=jnp.float32))
        coords_ref[...] = acc

    return _topk_body


def _topk_call(centers, centers_t, centers8, axes9, b0):
    return pl.pallas_call(
        _make_topk_body(b0),
        grid=(BH, L // R),
        in_specs=[
            pl.BlockSpec((None, R, 3), lambda b, r: (b, r, 0)),
            pl.BlockSpec((None, 3, L), lambda b, r: (b, 0, 0)),
            pl.BlockSpec((None, L, 8), lambda b, r: (b, 0, 0)),
            pl.BlockSpec((None, R, 9), lambda b, r: (b, r, 0)),
        ],
        out_specs=[
            pl.BlockSpec((None, R, 3 * K), lambda b, r: (b, r, 0)),
            pl.BlockSpec((None, R, K), lambda b, r: (b, r, 0)),
        ],
        out_shape=[
            jax.ShapeDtypeStruct((BH, L, 3 * K), jnp.float32),
            jax.ShapeDtypeStruct((BH, L, K), jnp.int32),
        ],
        scratch_shapes=[
            pltpu.VMEM((R, L), jnp.float32),
            pltpu.VMEM((R, 3 * K), jnp.float32),
        ],
    )(centers, centers_t, centers8, axes9)


@functools.cache
def _gather_kernel_build():
    @functools.partial(
        pl.kernel,
        mesh=plsc.VectorSubcoreMesh(core_axis_name="c", subcore_axis_name="s"),
        out_type=jax.ShapeDtypeStruct((TOTAL_H, D), jnp.float32),
        scratch_types=[
            pltpu.VMEM((ROWS_PER_W, IDX_COLS), jnp.int32),
            pltpu.VMEM((2, IDX_COLS, D), jnp.float32),
            pltpu.SemaphoreType.DMA,
        ],
    )
    def _gather_kernel(table_hbm, idx_hbm, out_hbm, idx_v, bufs, sem):
        wid = lax.axis_index("s") * NC + lax.axis_index("c")
        base = wid * ROWS_PER_W
        pltpu.sync_copy(idx_hbm.at[pl.ds(base, ROWS_PER_W)], idx_v)
        pltpu.async_copy(table_hbm.at[idx_v.at[0]], bufs.at[0], sem)

        def body(j, carry):
            jm = lax.rem(j, 2)
            pltpu.make_async_copy(
                table_hbm.at[pl.ds(0, IDX_COLS)], bufs.at[jm], sem).wait()
            nxt = j + 1

            @pl.when(nxt < ROWS_PER_W)
            def _():
                pltpu.async_copy(
                    table_hbm.at[idx_v.at[nxt]], bufs.at[lax.rem(nxt, 2)], sem)

            pltpu.sync_copy(
                bufs.at[jm], out_hbm.at[pl.ds((base + j) * IDX_COLS, IDX_COLS)])
            return carry

        lax.fori_loop(0, ROWS_PER_W, body, 0)

    return _gather_kernel


def kernel(frame, attr):
    centers = frame[:, :, 0, :]
    centers_t = jnp.transpose(centers, (0, 2, 1))
    centers8 = jnp.concatenate(
        [centers, jnp.zeros((B, L, 5), jnp.float32)], axis=-1)
    axes9 = frame[:, :, 1:4, :].reshape(B, L, 9)
    table = attr.reshape(B * L, D)
    gather = _gather_kernel_build()

    coords, nbr = _topk_call(centers, centers_t, centers8, axes9, 0)
    gathered = gather(table, nbr.reshape(IDX_ROWS_H, IDX_COLS))

    coords4 = coords.reshape(B, L, K, 3)
    attrs4 = gathered.reshape(B, L, K, D)
    return jnp.concatenate([coords4, attrs4], axis=-1)
